# Initial kernel scaffold; baseline (speedup 1.0000x reference)
import jax
import jax.numpy as jnp
from jax.experimental import pallas as pl


def kernel(x, edge_index, edge_attr, g1, b1, lin1_W, lin1_b, g3, b3, lin3_W, lin3_b, c1_Wx, c1_We, c1_asrc, c1_adst, c1_aedge, c1_b, c3_Wx, c3_We, c3_asrc, c3_adst, c3_aedge, c3_b):
    raise NotImplementedError("write your pallas kernel here")


# speedup vs baseline: 1.9046x; 1.9046x over previous
import jax
import jax.numpy as jnp
from jax.experimental import pallas as pl
from jax.experimental.pallas import tpu as pltpu

_NEG = -3.4e38


def _pick(n, prefs):
    for p in prefs:
        if n % p == 0:
            return p
    return n


def _stats_kernel(x_ref, o_ref):
    @pl.when(pl.program_id(0) == 0)
    def _():
        o_ref[...] = jnp.zeros_like(o_ref)

    xb = x_ref[...]
    o_ref[0:1, :] += jnp.sum(xb, 0, keepdims=True)
    o_ref[1:2, :] += jnp.sum(xb * xb, 0, keepdims=True)


def _colstats(x, rb):
    nb = x.shape[0] // rb
    return pl.pallas_call(
        _stats_kernel,
        grid=(nb,),
        in_specs=[pl.BlockSpec((rb, x.shape[1]), lambda i: (i, 0))],
        out_specs=pl.BlockSpec((2, x.shape[1]), lambda i: (0, 0)),
        out_shape=jax.ShapeDtypeStruct((2, x.shape[1]), jnp.float32),
    )(x)


def _dense1_kernel(x_ref, st_ref, g_ref, b_ref, W_ref, bias_ref, As_ref, Ad_ref,
                   h_ref, h0_ref, ss_ref, sd_ref, mx_ref, *, n):
    i = pl.program_id(0)
    m = st_ref[0:1, :] / n
    var = st_ref[1:2, :] / n - m * m
    inv = jax.lax.rsqrt(var + 1e-5)
    h = (x_ref[...] - m) * inv * g_ref[0:1, :] + b_ref[0:1, :]
    h_ref[...] = h
    h0_ref[...] = jnp.dot(h, W_ref[...], preferred_element_type=jnp.float32) + bias_ref[0:1, :]
    ss = jnp.dot(h, As_ref[...], preferred_element_type=jnp.float32)
    sd = jnp.dot(h, Ad_ref[...], preferred_element_type=jnp.float32)
    ss_ref[...] = ss
    sd_ref[...] = sd

    @pl.when(i == 0)
    def _():
        mx_ref[...] = jnp.full_like(mx_ref, _NEG)

    mx_ref[0:1, :] = jnp.maximum(mx_ref[0:1, :], jnp.max(ss, 0, keepdims=True))
    mx_ref[1:2, :] = jnp.maximum(mx_ref[1:2, :], jnp.max(sd, 0, keepdims=True))


def _edense_kernel(ef_ref, B1_ref, B2_ref, e1_ref, e2_ref, m1_ref, m2_ref):
    i = pl.program_id(0)
    ef = ef_ref[...]
    e1 = jnp.dot(ef, B1_ref[...], preferred_element_type=jnp.float32)
    e2 = jnp.dot(ef, B2_ref[...], preferred_element_type=jnp.float32)
    e1_ref[...] = e1
    e2_ref[...] = e2

    @pl.when(i == 0)
    def _():
        m1_ref[...] = jnp.full_like(m1_ref, _NEG)
        m2_ref[...] = jnp.full_like(m2_ref, _NEG)

    m1_ref[...] = jnp.maximum(m1_ref[...], jnp.max(e1, 0, keepdims=True))
    m2_ref[...] = jnp.maximum(m2_ref[...], jnp.max(e2, 0, keepdims=True))


def _den_kernel(src_ref, dst_ref, et_ref, ss_ref, sd_ref, ee_ref, mS_ref, mE_ref,
                ex_ref, den_ref, *, eblk, per_t):
    i = pl.program_id(0)

    @pl.when(i == 0)
    def _():
        den_ref[...] = jnp.zeros_like(den_ref)

    W = ee_ref.shape[1]
    B = mS_ref[0:1, :] + mS_ref[1:2, :] + mE_ref[0:1, :]
    B = jnp.where(B >= 0, B, 0.2 * B)
    lane_t = jax.lax.broadcasted_iota(jnp.int32, (1, W), 1) // per_t

    def body(j, carry):
        s = src_ref[0, 0, j]
        d = dst_ref[0, 0, j]
        t = et_ref[0, 0, j]
        row = ss_ref[pl.ds(s, 1), :] + sd_ref[pl.ds(d, 1), :] + ee_ref[pl.ds(j, 1), :]
        row = jnp.where(row >= 0, row, 0.2 * row)
        ex = jnp.where(lane_t == t, jnp.exp(row - B), 0.0)
        ex_ref[pl.ds(j, 1), :] = ex
        den_ref[pl.ds(d, 1), :] += ex
        return carry

    jax.lax.fori_loop(0, eblk, body, 0)


def _agg1_kernel(src_ref, dst_ref, et_ref, ex_ref, den_ref, h_ref, agg_ref,
                 *, eblk, t):
    i = pl.program_id(0)

    @pl.when(i == 0)
    def _():
        agg_ref[...] = jnp.zeros_like(agg_ref)

    def body(j, carry):
        tt = et_ref[0, 0, j]

        @pl.when(tt == t)
        def _():
            s = src_ref[0, 0, j]
            d = dst_ref[0, 0, j]
            ar = ex_ref[pl.ds(j, 1), :] / (den_ref[pl.ds(d, 1), :] + 1e-30)
            hr = h_ref[pl.ds(s, 1), :]
            for hh in range(4):
                agg_ref[hh, pl.ds(d, 1), :] += ar[0, hh] * hr

        return carry

    jax.lax.fori_loop(0, eblk, body, 0)


def _post1_kernel(h0_ref, a0_ref, a1_ref, a2_ref, a3_ref,
                  w0_ref, w1_ref, w2_ref, w3_ref, h1r_ref, st2_ref):
    i = pl.program_id(1)
    res = h0_ref[...]
    for a_ref, w_ref in ((a0_ref, w0_ref), (a1_ref, w1_ref),
                         (a2_ref, w2_ref), (a3_ref, w3_ref)):
        res = res + jnp.dot(a_ref[...][0], w_ref[...][0],
                            preferred_element_type=jnp.float32)
    r = jnp.maximum(res, 0.0)
    h1r_ref[...] = r

    @pl.when(i == 0)
    def _():
        st2_ref[...] = jnp.zeros_like(st2_ref)

    st2_ref[0:1, :] += jnp.sum(r, 0, keepdims=True)
    st2_ref[1:2, :] += jnp.sum(r * r, 0, keepdims=True)


def _dense2_kernel(h1_ref, st_ref, g_ref, b_ref, W3_ref, bias_ref, S2_ref, D2_ref,
                   Wc_ref, o0_ref, ss_ref, sd_ref, xp_ref, mx_ref, *, n):
    i = pl.program_id(0)
    m = st_ref[0:1, :] / n
    var = st_ref[1:2, :] / n - m * m
    inv = jax.lax.rsqrt(var + 1e-5)
    h2 = (h1_ref[...] - m) * inv * g_ref[0:1, :] + b_ref[0:1, :]
    o0_ref[...] = jnp.dot(h2, W3_ref[...], preferred_element_type=jnp.float32) + bias_ref[0:1, :]
    ss = jnp.dot(h2, S2_ref[...], preferred_element_type=jnp.float32)
    sd = jnp.dot(h2, D2_ref[...], preferred_element_type=jnp.float32)
    ss_ref[...] = ss
    sd_ref[...] = sd
    xp_ref[...] = jnp.dot(h2, Wc_ref[...], preferred_element_type=jnp.float32)

    @pl.when(i == 0)
    def _():
        mx_ref[...] = jnp.full_like(mx_ref, _NEG)

    mx_ref[0:1, :] = jnp.maximum(mx_ref[0:1, :], jnp.max(ss, 0, keepdims=True))
    mx_ref[1:2, :] = jnp.maximum(mx_ref[1:2, :], jnp.max(sd, 0, keepdims=True))


def _agg2_kernel(src_ref, dst_ref, et_ref, ex_ref, den_ref, xp_ref, o_ref, *, eblk):
    i = pl.program_id(0)

    @pl.when(i == 0)
    def _():
        o_ref[...] = jnp.zeros_like(o_ref)

    lane2 = jax.lax.broadcasted_iota(jnp.int32, (1, 8), 1) // 2

    def body(j, carry):
        s = src_ref[0, 0, j]
        d = dst_ref[0, 0, j]
        t = et_ref[0, 0, j]
        ar = ex_ref[pl.ds(j, 1), :] / (den_ref[pl.ds(d, 1), :] + 1e-30)
        a = jnp.sum(ar)
        xr = xp_ref[pl.ds(s, 1), :]
        o_ref[pl.ds(d, 1), :] += jnp.where(lane2 == t, xr, 0.0) * a
        return carry

    jax.lax.fori_loop(0, eblk, body, 0)


def _final_kernel(o0_ref, ow_ref, out_ref):
    ow = ow_ref[...]
    acc = o0_ref[...] + ow[:, 0:2] + ow[:, 2:4] + ow[:, 4:6] + ow[:, 6:8]
    out_ref[...] = jnp.maximum(acc, 0.0)


def kernel(x, edge_index, edge_attr, g1, b1, lin1_W, lin1_b, g3, b3, lin3_W, lin3_b,
           c1_Wx, c1_We, c1_asrc, c1_adst, c1_aedge, c1_b,
           c3_Wx, c3_We, c3_asrc, c3_adst, c3_aedge, c3_b):
    N, F = x.shape
    E = edge_index.shape[1]
    D = edge_attr.shape[1] - 1
    T, _, HF = c1_Wx.shape
    H = c1_asrc.shape[1]
    C = HF // H
    f32 = jnp.float32

    RB = _pick(N, [1000, 400, 200, 80, 8])
    EBLK = _pick(E, [8000, 4000, 2000, 1000, 500])
    nb, nbE = N // RB, E // EBLK

    src3 = edge_index[0].reshape(nbE, 1, EBLK)
    dst3 = edge_index[1].reshape(nbE, 1, EBLK)
    et3 = edge_attr[:, -1].astype(jnp.int32).reshape(nbE, 1, EBLK)
    ef = edge_attr[:, :-1]

    Wx4 = c1_Wx.reshape(T, F, H, C)
    A_src = jnp.einsum('tfhc,thc->fth', Wx4, c1_asrc).reshape(F, T * H)
    A_dst = jnp.einsum('tfhc,thc->fth', Wx4, c1_adst).reshape(F, T * H)
    B1 = jnp.einsum('tdhc,thc->dth', c1_We.reshape(T, D, H, C), c1_aedge).reshape(D, T * H)
    bias1 = (lin1_b + c1_b.sum(0)).reshape(1, HF)
    Whead = c1_Wx.reshape(T, F, H, C).transpose(0, 2, 1, 3)
    S2 = jnp.einsum('tfk,tk->ft', c3_Wx, c3_asrc[:, 0, :])
    D2 = jnp.einsum('tfk,tk->ft', c3_Wx, c3_adst[:, 0, :])
    B2 = jnp.einsum('tdk,tk->dt', c3_We, c3_aedge[:, 0, :])
    W2cat = c3_Wx.transpose(1, 0, 2).reshape(HF, T * 2)
    bias2 = (lin3_b + c3_b.sum(0)).reshape(1, 2)

    st1 = _colstats(x, RB)

    h, h0, s_src, s_dst, mxS1 = pl.pallas_call(
        lambda *a: _dense1_kernel(*a, n=float(N)),
        grid=(nb,),
        in_specs=[
            pl.BlockSpec((RB, F), lambda i: (i, 0)),
            pl.BlockSpec((2, F), lambda i: (0, 0)),
            pl.BlockSpec((1, F), lambda i: (0, 0)),
            pl.BlockSpec((1, F), lambda i: (0, 0)),
            pl.BlockSpec((F, HF), lambda i: (0, 0)),
            pl.BlockSpec((1, HF), lambda i: (0, 0)),
            pl.BlockSpec((F, T * H), lambda i: (0, 0)),
            pl.BlockSpec((F, T * H), lambda i: (0, 0)),
        ],
        out_specs=[
            pl.BlockSpec((RB, F), lambda i: (i, 0)),
            pl.BlockSpec((RB, HF), lambda i: (i, 0)),
            pl.BlockSpec((RB, T * H), lambda i: (i, 0)),
            pl.BlockSpec((RB, T * H), lambda i: (i, 0)),
            pl.BlockSpec((2, T * H), lambda i: (0, 0)),
        ],
        out_shape=[
            jax.ShapeDtypeStruct((N, F), f32),
            jax.ShapeDtypeStruct((N, HF), f32),
            jax.ShapeDtypeStruct((N, T * H), f32),
            jax.ShapeDtypeStruct((N, T * H), f32),
            jax.ShapeDtypeStruct((2, T * H), f32),
        ],
    )(x, st1, g1.reshape(1, F), b1.reshape(1, F), lin1_W, bias1, A_src, A_dst)

    et1, et2, mxE1, mxE2 = pl.pallas_call(
        _edense_kernel,
        grid=(nbE,),
        in_specs=[
            pl.BlockSpec((EBLK, D), lambda i: (i, 0)),
            pl.BlockSpec((D, T * H), lambda i: (0, 0)),
            pl.BlockSpec((D, T), lambda i: (0, 0)),
        ],
        out_specs=[
            pl.BlockSpec((EBLK, T * H), lambda i: (i, 0)),
            pl.BlockSpec((EBLK, T), lambda i: (i, 0)),
            pl.BlockSpec((1, T * H), lambda i: (0, 0)),
            pl.BlockSpec((1, T), lambda i: (0, 0)),
        ],
        out_shape=[
            jax.ShapeDtypeStruct((E, T * H), f32),
            jax.ShapeDtypeStruct((E, T), f32),
            jax.ShapeDtypeStruct((1, T * H), f32),
            jax.ShapeDtypeStruct((1, T), f32),
        ],
    )(ef, B1, B2)

    def _den_call(ss, sd, ee, mS, mE, width, per_t):
        return pl.pallas_call(
            lambda *a: _den_kernel(*a, eblk=EBLK, per_t=per_t),
            grid=(nbE,),
            in_specs=[
                pl.BlockSpec((1, 1, EBLK), lambda i: (i, 0, 0), memory_space=pltpu.SMEM),
                pl.BlockSpec((1, 1, EBLK), lambda i: (i, 0, 0), memory_space=pltpu.SMEM),
                pl.BlockSpec((1, 1, EBLK), lambda i: (i, 0, 0), memory_space=pltpu.SMEM),
                pl.BlockSpec((N, width), lambda i: (0, 0)),
                pl.BlockSpec((N, width), lambda i: (0, 0)),
                pl.BlockSpec((EBLK, width), lambda i: (i, 0)),
                pl.BlockSpec((2, width), lambda i: (0, 0)),
                pl.BlockSpec((1, width), lambda i: (0, 0)),
            ],
            out_specs=[
                pl.BlockSpec((EBLK, width), lambda i: (i, 0)),
                pl.BlockSpec((N, width), lambda i: (0, 0)),
            ],
            out_shape=[
                jax.ShapeDtypeStruct((E, width), f32),
                jax.ShapeDtypeStruct((N, width), f32),
            ],
        )(src3, dst3, et3, ss, sd, ee, mS, mE)

    ex1, den1 = _den_call(s_src, s_dst, et1, mxS1, mxE1, T * H, H)

    aggs = []
    for t in range(T):
        aggs.append(pl.pallas_call(
            lambda *a, _t=t: _agg1_kernel(*a, eblk=EBLK, t=_t),
            grid=(nbE,),
            in_specs=[
                pl.BlockSpec((1, 1, EBLK), lambda i: (i, 0, 0), memory_space=pltpu.SMEM),
                pl.BlockSpec((1, 1, EBLK), lambda i: (i, 0, 0), memory_space=pltpu.SMEM),
                pl.BlockSpec((1, 1, EBLK), lambda i: (i, 0, 0), memory_space=pltpu.SMEM),
                pl.BlockSpec((EBLK, H), lambda i: (i, 0)),
                pl.BlockSpec((N, H), lambda i: (0, 0)),
                pl.BlockSpec((N, F), lambda i: (0, 0)),
            ],
            out_specs=pl.BlockSpec((H, N, F), lambda i: (0, 0, 0)),
            out_shape=jax.ShapeDtypeStruct((H, N, F), f32),
            compiler_params=pltpu.CompilerParams(vmem_limit_bytes=100 * 1024 * 1024),
        )(src3, dst3, et3, ex1[:, H * t:H * (t + 1)], den1[:, H * t:H * (t + 1)], h))

    h1r, st2 = pl.pallas_call(
        _post1_kernel,
        grid=(H, nb),
        in_specs=[pl.BlockSpec((RB, C), lambda hh, i: (i, hh))] + [
            pl.BlockSpec((1, RB, F), lambda hh, i: (hh, i, 0)) for _ in range(T)
        ] + [
            pl.BlockSpec((1, F, C), lambda hh, i: (hh, 0, 0)) for _ in range(T)
        ],
        out_specs=[
            pl.BlockSpec((RB, C), lambda hh, i: (i, hh)),
            pl.BlockSpec((2, C), lambda hh, i: (0, hh)),
        ],
        out_shape=[
            jax.ShapeDtypeStruct((N, HF), f32),
            jax.ShapeDtypeStruct((2, HF), f32),
        ],
    )(h0, *aggs, *[Whead[t] for t in range(T)])

    o0, s2s, s2d, xp2, mxS2 = pl.pallas_call(
        lambda *a: _dense2_kernel(*a, n=float(N)),
        grid=(nb,),
        in_specs=[
            pl.BlockSpec((RB, HF), lambda i: (i, 0)),
            pl.BlockSpec((2, HF), lambda i: (0, 0)),
            pl.BlockSpec((1, HF), lambda i: (0, 0)),
            pl.BlockSpec((1, HF), lambda i: (0, 0)),
            pl.BlockSpec((HF, 2), lambda i: (0, 0)),
            pl.BlockSpec((1, 2), lambda i: (0, 0)),
            pl.BlockSpec((HF, T), lambda i: (0, 0)),
            pl.BlockSpec((HF, T), lambda i: (0, 0)),
            pl.BlockSpec((HF, T * 2), lambda i: (0, 0)),
        ],
        out_specs=[
            pl.BlockSpec((RB, 2), lambda i: (i, 0)),
            pl.BlockSpec((RB, T), lambda i: (i, 0)),
            pl.BlockSpec((RB, T), lambda i: (i, 0)),
            pl.BlockSpec((RB, T * 2), lambda i: (i, 0)),
            pl.BlockSpec((2, T), lambda i: (0, 0)),
        ],
        out_shape=[
            jax.ShapeDtypeStruct((N, 2), f32),
            jax.ShapeDtypeStruct((N, T), f32),
            jax.ShapeDtypeStruct((N, T), f32),
            jax.ShapeDtypeStruct((N, T * 2), f32),
            jax.ShapeDtypeStruct((2, T), f32),
        ],
    )(h1r, st2, g3.reshape(1, HF), b3.reshape(1, HF), lin3_W, bias2, S2, D2, W2cat)

    ex2, den2 = _den_call(s2s, s2d, et2, mxS2, mxE2, T, 1)

    ow = pl.pallas_call(
        lambda *a: _agg2_kernel(*a, eblk=EBLK),
        grid=(nbE,),
        in_specs=[
            pl.BlockSpec((1, 1, EBLK), lambda i: (i, 0, 0), memory_space=pltpu.SMEM),
            pl.BlockSpec((1, 1, EBLK), lambda i: (i, 0, 0), memory_space=pltpu.SMEM),
            pl.BlockSpec((1, 1, EBLK), lambda i: (i, 0, 0), memory_space=pltpu.SMEM),
            pl.BlockSpec((EBLK, T), lambda i: (i, 0)),
            pl.BlockSpec((N, T), lambda i: (0, 0)),
            pl.BlockSpec((N, T * 2), lambda i: (0, 0)),
        ],
        out_specs=pl.BlockSpec((N, T * 2), lambda i: (0, 0)),
        out_shape=jax.ShapeDtypeStruct((N, T * 2), f32),
    )(src3, dst3, et3, ex2, den2, xp2)

    return pl.pallas_call(
        _final_kernel,
        grid=(nb,),
        in_specs=[
            pl.BlockSpec((RB, 2), lambda i: (i, 0)),
            pl.BlockSpec((RB, T * 2), lambda i: (i, 0)),
        ],
        out_specs=pl.BlockSpec((RB, 2), lambda i: (i, 0)),
        out_shape=jax.ShapeDtypeStruct((N, 2), f32),
    )(o0, ow)
